# baseline (speedup 1.0000x reference)
import functools

import jax
import jax.numpy as jnp
from jax import lax
from jax.experimental import pallas as pl
from jax.experimental.pallas import tpu as pltpu
from jax.experimental.pallas import tpu_sc as plsc

KS = 5
DIM = 3
K = KS ** DIM
N = 6890
E = 41328
NPAD = 6912
EPAD = 41472
P = 8 * EPAD
BLK = 256
NW = 32
SCH = 1024
NSUP = 12
CAP = NW * NSUP * SCH
NBLK = CAP // BLK
ROWS_PER_TILE = NPAD // 16

_MESH = dict(core_axis_name="c", subcore_axis_name="s")
_SC_PARAMS = pltpu.CompilerParams(use_tc_tiling_on_sc=False)


def _basis(pseudo8):
    def body(ps_ref, b_ref, k_ref):
        pid = pl.program_id(0)
        v = ps_ref[...] * float(KS - 1)
        bot = jnp.floor(v)
        frac = v - bot
        boti = bot.astype(jnp.int32)
        e_idx = pid * 512 + lax.broadcasted_iota(jnp.int32, (512, 1), 0)
        valid = (e_idx < E).astype(jnp.float32)
        bs, ks = [], []
        for s in range(8):
            b = valid
            kidx = jnp.zeros((512, 1), jnp.int32)
            stride = 1
            for d in range(DIM):
                o = (s >> d) & 1
                fd = frac[:, d:d + 1]
                b = b * (fd if o == 1 else (1.0 - fd))
                kd = jnp.clip(boti[:, d:d + 1] + o, 0, KS - 1)
                kidx = kidx + kd * stride
                stride *= KS
            bs.append(b)
            ks.append(kidx)
        b_ref[...] = jnp.concatenate(bs, axis=1)
        k_ref[...] = jnp.concatenate(ks, axis=1)

    return pl.pallas_call(
        body,
        grid=(EPAD // 512,),
        in_specs=[pl.BlockSpec((512, 8), lambda j: (j, 0))],
        out_specs=[pl.BlockSpec((512, 8), lambda j: (j, 0)),
                   pl.BlockSpec((512, 8), lambda j: (j, 0))],
        out_shape=[jax.ShapeDtypeStruct((EPAD, 8), jnp.float32),
                   jax.ShapeDtypeStruct((EPAD, 8), jnp.int32)],
    )(pseudo8)


def _routing(kk, bb, ss, dd):
    order = jnp.argsort(kk)
    kk_s = kk[order]
    bb_s = bb[order]
    ss_s = ss[order]
    dd_s = dd[order]
    off = jnp.searchsorted(kk_s, jnp.arange(K + 1, dtype=jnp.int32),
                           side='left').astype(jnp.int32)
    counts = off[1:] - off[:-1]
    pc = ((counts + BLK - 1) // BLK) * BLK
    pend = jnp.cumsum(pc).astype(jnp.int32)
    poff = pend - pc
    q = jnp.arange(CAP, dtype=jnp.int32)
    kq = jnp.searchsorted(pend, q, side='right').astype(jnp.int32)
    kqc = jnp.minimum(kq, K - 1)
    r = q - poff[kqc]
    valid = (kq < K) & (r < counts[kqc])
    i = jnp.clip(off[kqc] + r, 0, P - 1)
    src_cap = jnp.where(valid, ss_s[i], 0).astype(jnp.int32)
    bb_cap = jnp.where(valid, bb_s[i], 0.0)
    dst_cap = jnp.where(valid, dd_s[i], 0).astype(jnp.int32)
    blkk = jnp.minimum(
        jnp.searchsorted(pend, jnp.arange(NBLK, dtype=jnp.int32) * BLK,
                         side='right'), K - 1).astype(jnp.int32)
    return src_cap, bb_cap, dst_cap, blkk


def _sc_gather(table, idx2, D):
    mesh = plsc.VectorSubcoreMesh(**_MESH)

    @functools.partial(
        pl.kernel,
        out_type=jax.ShapeDtypeStruct((CAP, D), jnp.float32),
        mesh=mesh,
        compiler_params=_SC_PARAMS,
        scratch_types=[pltpu.VMEM((8, 128), jnp.int32),
                       pltpu.VMEM((SCH, D), jnp.float32),
                       pltpu.SemaphoreType.DMA],
        name=f"sc_gather_{D}",
    )
    def k(table_hbm, idx_hbm, out_hbm, idx_v, rows_v, sem):
        c = lax.axis_index("c")
        s = lax.axis_index("s")
        wid = s * 2 + c

        def body(ci, carry):
            base = pl.multiple_of((wid * NSUP + ci) * SCH, SCH)
            pltpu.sync_copy(idx_hbm.at[pl.ds(pl.multiple_of(base // 128, 8), 8)],
                            idx_v)
            cps = [pltpu.async_copy(table_hbm.at[idx_v.at[j]],
                                    rows_v.at[pl.ds(j * 128, 128)], sem)
                   for j in range(8)]
            for cp in cps:
                cp.wait()
            pltpu.sync_copy(rows_v, out_hbm.at[pl.ds(base, SCH)])
            return carry

        lax.fori_loop(0, NSUP, body, 0)

    return k(table, idx2)


def _sc_scatter(op, idx2, D, zeros_tbl):
    mesh = plsc.VectorSubcoreMesh(**_MESH)

    @functools.partial(
        pl.kernel,
        out_type=jax.ShapeDtypeStruct((2, NPAD, D), jnp.float32),
        mesh=mesh,
        compiler_params=_SC_PARAMS,
        scratch_types=[pltpu.VMEM((8, 128), jnp.int32),
                       pltpu.VMEM((SCH, D), jnp.float32),
                       pltpu.VMEM_SHARED((NPAD, D), jnp.float32)],
        name=f"sc_scatter_{D}",
    )
    def k(op_hbm, idx_hbm, zeros_hbm, out_hbm, idx_v, rows_v, acc_sh):
        c = lax.axis_index("c")
        s = lax.axis_index("s")
        row0 = s * ROWS_PER_TILE
        pltpu.sync_copy(zeros_hbm.at[pl.ds(row0, ROWS_PER_TILE)],
                        acc_sh.at[pl.ds(row0, ROWS_PER_TILE)])
        plsc.subcore_barrier()

        def body(ci, carry):
            base = pl.multiple_of(((c * 16 + s) * NSUP + ci) * SCH, SCH)
            pltpu.sync_copy(idx_hbm.at[pl.ds(pl.multiple_of(base // 128, 8), 8)],
                            idx_v)
            pltpu.sync_copy(op_hbm.at[pl.ds(base, SCH)], rows_v)
            for j in range(8):
                pltpu.sync_copy(rows_v.at[pl.ds(j * 128, 128)],
                                acc_sh.at[idx_v.at[j]], add=True)
            return carry

        lax.fori_loop(0, NSUP, body, 0)
        plsc.subcore_barrier()
        pltpu.sync_copy(acc_sh.at[pl.ds(row0, ROWS_PER_TILE)],
                        out_hbm.at[c, pl.ds(row0, ROWS_PER_TILE)])

    return k(op, idx2, zeros_tbl)


def _tc_bucket_matmul(gp, bb2, W, blkk, Din, Dout):
    def body(bk_ref, gp_ref, bb_ref, w_ref, op_ref):
        x = gp_ref[...] * bb_ref[...]
        op_ref[...] = jnp.dot(x, w_ref[0],
                              preferred_element_type=jnp.float32)

    grid_spec = pltpu.PrefetchScalarGridSpec(
        num_scalar_prefetch=1,
        grid=(NBLK,),
        in_specs=[pl.BlockSpec((BLK, Din), lambda j, bk: (j, 0)),
                  pl.BlockSpec((BLK, 1), lambda j, bk: (j, 0)),
                  pl.BlockSpec((1, Din, Dout), lambda j, bk: (bk[j], 0, 0))],
        out_specs=pl.BlockSpec((BLK, Dout), lambda j, bk: (j, 0)),
    )
    return pl.pallas_call(
        body,
        grid_spec=grid_spec,
        out_shape=jax.ShapeDtypeStruct((CAP, Dout), jnp.float32),
    )(blkk, gp, bb2, W)


def _tc_post(part, degp, xt, root, bias, Din, Dout):
    def body(p_ref, d_ref, x_ref, r_ref, b_ref, o_ref):
        spl = p_ref[0] + p_ref[1]
        deg = jnp.maximum(d_ref[0, :, 0:1] + d_ref[1, :, 0:1], 1.0)
        val = spl / deg + jnp.dot(x_ref[...], r_ref[...],
                                  preferred_element_type=jnp.float32) + b_ref[...]
        o_ref[...] = jnp.where(val > 0, val,
                               jnp.exp(jnp.minimum(val, 0.0)) - 1.0)

    return pl.pallas_call(
        body,
        grid=(NPAD // 256,),
        in_specs=[pl.BlockSpec((2, 256, Dout), lambda j: (0, j, 0)),
                  pl.BlockSpec((2, 256, 16), lambda j: (0, j, 0)),
                  pl.BlockSpec((256, Din), lambda j: (j, 0)),
                  pl.BlockSpec((Din, Dout), lambda j: (0, 0)),
                  pl.BlockSpec((1, Dout), lambda j: (0, 0))],
        out_specs=pl.BlockSpec((256, Dout), lambda j: (j, 0)),
        out_shape=jax.ShapeDtypeStruct((NPAD, Dout), jnp.float32),
    )(part, degp, xt, root, bias)


def _tc_head(h, lw1, lb1, lw2p, lb2p):
    def body(h_ref, w1_ref, b1_ref, w2_ref, b2_ref, o_ref):
        t = jnp.dot(h_ref[...], w1_ref[...],
                    preferred_element_type=jnp.float32) + b1_ref[...]
        t = jnp.where(t > 0, t, jnp.exp(jnp.minimum(t, 0.0)) - 1.0)
        lg = jnp.dot(t, w2_ref[...],
                     preferred_element_type=jnp.float32) + b2_ref[...]
        col = lax.broadcasted_iota(jnp.int32, (256, NPAD), 1)
        lg = jnp.where(col < N, lg, -1e30)
        m = jnp.max(lg, axis=1, keepdims=True)
        z = lg - m
        lse = jnp.log(jnp.sum(jnp.exp(z), axis=1, keepdims=True))
        o_ref[...] = z - lse

    return pl.pallas_call(
        body,
        grid=(NPAD // 256,),
        in_specs=[pl.BlockSpec((256, 64), lambda j: (j, 0)),
                  pl.BlockSpec((64, 256), lambda j: (0, 0)),
                  pl.BlockSpec((1, 256), lambda j: (0, 0)),
                  pl.BlockSpec((256, NPAD), lambda j: (0, 0)),
                  pl.BlockSpec((1, NPAD), lambda j: (0, 0))],
        out_specs=pl.BlockSpec((256, NPAD), lambda j: (j, 0)),
        out_shape=jax.ShapeDtypeStruct((NPAD, NPAD), jnp.float32),
    )(h, lw1, lb1, lw2p, lb2p)


def kernel(x, edge_index, pseudo, W1, r1, b1, W2, r2, b2, W3, r3, b3,
           W4, r4, b4, W5, r5, b5, W6, r6, b6, lw1, lb1, lw2, lb2):
    src = edge_index[0].astype(jnp.int32)
    dst = edge_index[1].astype(jnp.int32)
    src_p = jnp.pad(src, (0, EPAD - E))
    dst_p = jnp.pad(dst, (0, EPAD - E))
    pseudo8 = jnp.pad(pseudo.astype(jnp.float32), ((0, EPAD - E), (0, 5)))

    b8, k8 = _basis(pseudo8)
    kk = k8.reshape(-1)
    bb = b8.reshape(-1)
    ss = jnp.broadcast_to(src_p[:, None], (EPAD, 8)).reshape(-1)
    dd = jnp.broadcast_to(dst_p[:, None], (EPAD, 8)).reshape(-1)
    src_cap, bb_cap, dst_cap, blkk = _routing(kk, bb, ss, dd)

    src2 = src_cap.reshape(CAP // 128, 128)
    dst2 = dst_cap.reshape(CAP // 128, 128)
    bb2 = bb_cap.reshape(CAP, 1)

    zeros16 = jnp.zeros((NPAD, 16), jnp.float32)
    zeros32 = jnp.zeros((NPAD, 32), jnp.float32)
    zeros64 = jnp.zeros((NPAD, 64), jnp.float32)
    zeros_by_d = {16: zeros16, 32: zeros32, 64: zeros64}

    bbw = jnp.zeros((CAP, 16), jnp.float32).at[:, 0].set(bb_cap)
    degp = _sc_scatter(bbw, dst2, 16, zeros16)

    def layer(xt, W, root, bias, Din, Dout):
        gp = _sc_gather(xt, src2, Din)
        op = _tc_bucket_matmul(gp, bb2, W, blkk, Din, Dout)
        part = _sc_scatter(op, dst2, Dout, zeros_by_d[Dout])
        return _tc_post(part, degp, xt, root, bias.reshape(1, Dout),
                        Din, Dout)

    x1 = jnp.zeros((NPAD, 16), jnp.float32).at[:N, 0].set(x[:, 0])
    W1p = jnp.zeros((K, 16, 32), jnp.float32).at[:, :1, :].set(W1)
    r1p = jnp.zeros((16, 32), jnp.float32).at[0:1].set(r1)

    h = layer(x1, W1p, r1p, b1, 16, 32)
    h = layer(h, W2, r2, b2, 32, 64)
    h = layer(h, W3, r3, b3, 64, 64)
    h = layer(h, W4, r4, b4, 64, 64)
    h = layer(h, W5, r5, b5, 64, 64)
    h = layer(h, W6, r6, b6, 64, 64)

    lw2p = jnp.pad(lw2, ((0, 0), (0, NPAD - N)))
    lb2p = jnp.pad(lb2, (0, NPAD - N)).reshape(1, NPAD)
    out_full = _tc_head(h, lw1, lb1.reshape(1, 256), lw2p, lb2p)
    return out_full[:N, :N]


# speedup vs baseline: 2.7482x; 2.7482x over previous
import functools

import jax
import jax.numpy as jnp
from jax import lax
from jax.experimental import pallas as pl
from jax.experimental.pallas import tpu as pltpu
from jax.experimental.pallas import tpu_sc as plsc

KS = 5
DIM = 3
K = KS ** DIM
N = 6890
E = 41328
NPAD = 6912
EPAD = 41472
EB = 512
NEB = EPAD // EB
P = 8 * EPAD
BLK = 256
NW = 32
SCH = 1024
NSUP = 12
CAP = NW * NSUP * SCH
NBLK = CAP // BLK
ROWS_PER_TILE = NPAD // 16

_MESH = dict(core_axis_name="c", subcore_axis_name="s")
_SC_PARAMS = pltpu.CompilerParams(use_tc_tiling_on_sc=False,
                                  needs_layout_passes=False)


def _basis(pseudo8):
    def body(ps_ref, b_ref, k_ref):
        pid = pl.program_id(0)
        v = ps_ref[...] * float(KS - 1)
        bot = jnp.floor(v)
        frac = v - bot
        boti = bot.astype(jnp.int32)
        e_idx = pid * EB + lax.broadcasted_iota(jnp.int32, (EB, 1), 0)
        valid = (e_idx < E).astype(jnp.float32)
        bs, ks = [], []
        for s in range(8):
            b = valid
            kidx = jnp.zeros((EB, 1), jnp.int32)
            stride = 1
            for d in range(DIM):
                o = (s >> d) & 1
                fd = frac[:, d:d + 1]
                b = b * (fd if o == 1 else (1.0 - fd))
                kd = jnp.clip(boti[:, d:d + 1] + o, 0, KS - 1)
                kidx = kidx + kd * stride
                stride *= KS
            bs.append(b)
            ks.append(kidx)
        b_ref[...] = jnp.concatenate(bs, axis=1)
        k_ref[...] = jnp.concatenate(ks, axis=1)

    return pl.pallas_call(
        body,
        grid=(NEB,),
        in_specs=[pl.BlockSpec((EB, 8), lambda j: (j, 0))],
        out_specs=[pl.BlockSpec((EB, 8), lambda j: (j, 0)),
                   pl.BlockSpec((EB, 8), lambda j: (j, 0))],
        out_shape=[jax.ShapeDtypeStruct((EPAD, 8), jnp.float32),
                   jax.ShapeDtypeStruct((EPAD, 8), jnp.int32)],
    )(pseudo8)


def _hist(k8):
    def body(k_ref, h_ref):
        iota_k = lax.broadcasted_iota(jnp.int32, (1, 128), 1)
        tot = jnp.zeros((1, 128), jnp.int32)
        for s in range(8):
            ks = k_ref[:, s:s + 1]
            oh = (ks == iota_k).astype(jnp.int32)
            tot = tot + jnp.sum(oh, axis=0, keepdims=True)
        h_ref[...] = tot.reshape(1, 1, 128)

    return pl.pallas_call(
        body,
        grid=(NEB,),
        in_specs=[pl.BlockSpec((EB, 8), lambda j: (j, 0))],
        out_specs=pl.BlockSpec((1, 1, 128), lambda j: (j, 0, 0)),
        out_shape=jax.ShapeDtypeStruct((NEB, 1, 128), jnp.int32),
    )(k8)


def _slots_records(k8, b8, srcf, dstf, poff2, blockpre):
    def body(k_ref, b_ref, s_ref, d_ref, po_ref, bp_ref, sl_ref, rc_ref):
        iota_k = lax.broadcasted_iota(jnp.int32, (1, 128), 1)
        base = po_ref[...] + bp_ref[0]
        run = jnp.zeros((1, 128), jnp.int32)
        slot_cols = []
        for s in range(8):
            ks = k_ref[:, s:s + 1]
            oh = (ks == iota_k).astype(jnp.int32)
            c = oh
            sh = 1
            while sh < EB:
                c = c + jnp.concatenate(
                    [jnp.zeros((sh, 128), jnp.int32), c[:EB - sh]], axis=0)
                sh *= 2
            excl = c - oh
            slot_s = jnp.sum(oh * (excl + run + base), axis=1, keepdims=True)
            slot_cols.append(slot_s)
            run = run + jnp.sum(oh, axis=0, keepdims=True)
        sl_ref[...] = jnp.concatenate(slot_cols, axis=1)

        sb = jnp.broadcast_to(s_ref[...], (EB, 8))
        db = jnp.broadcast_to(d_ref[...], (EB, 8))
        rec = jnp.concatenate(
            [sb[..., None], db[..., None], b_ref[...][..., None],
             jnp.zeros((EB, 8, 13), jnp.float32)], axis=2)
        rc_ref[...] = rec.reshape(EB * 8, 16)

    return pl.pallas_call(
        body,
        grid=(NEB,),
        in_specs=[pl.BlockSpec((EB, 8), lambda j: (j, 0)),
                  pl.BlockSpec((EB, 8), lambda j: (j, 0)),
                  pl.BlockSpec((EB, 1), lambda j: (j, 0)),
                  pl.BlockSpec((EB, 1), lambda j: (j, 0)),
                  pl.BlockSpec((1, 128), lambda j: (0, 0)),
                  pl.BlockSpec((1, 1, 128), lambda j: (j, 0, 0))],
        out_specs=[pl.BlockSpec((EB, 8), lambda j: (j, 0)),
                   pl.BlockSpec((EB * 8, 16), lambda j: (j, 0))],
        out_shape=[jax.ShapeDtypeStruct((EPAD, 8), jnp.int32),
                   jax.ShapeDtypeStruct((P, 16), jnp.float32)],
    )(k8, b8, srcf, dstf, poff2, blockpre)


def _rc_scatter(recs, slot2):
    mesh = plsc.VectorSubcoreMesh(**_MESH)
    nsup_p = P // SCH
    per_w = nsup_p // NW
    rem = nsup_p - per_w * NW

    @functools.partial(
        pl.kernel,
        out_type=jax.ShapeDtypeStruct((CAP, 16), jnp.float32),
        mesh=mesh,
        compiler_params=_SC_PARAMS,
        scratch_types=[pltpu.VMEM((8, 128), jnp.int32),
                       pltpu.VMEM((SCH, 16), jnp.float32),
                       pltpu.SemaphoreType.DMA],
        name="sc_rc_scatter",
    )
    def k(recs_hbm, slot_hbm, out_hbm, idx_v, rows_v, sem):
        c = lax.axis_index("c")
        s = lax.axis_index("s")
        wid = s * 2 + c

        def one(sc):
            base = pl.multiple_of(sc * SCH, SCH)
            pltpu.sync_copy(slot_hbm.at[pl.ds(pl.multiple_of(sc * 8, 8), 8)],
                            idx_v)
            pltpu.sync_copy(recs_hbm.at[pl.ds(base, SCH)], rows_v)
            cps = [pltpu.async_copy(rows_v.at[pl.ds(j * 128, 128)],
                                    out_hbm.at[idx_v.at[j]], sem)
                   for j in range(8)]
            for cp in cps:
                cp.wait()

        def body(i, carry):
            one(wid + NW * i)
            return carry

        lax.fori_loop(0, per_w, body, 0)

        @pl.when(wid < rem)
        def _():
            one(NW * per_w + wid)

    return k(recs, slot2)


def _extract(rc):
    mesh = plsc.VectorSubcoreMesh(**_MESH)

    @functools.partial(
        pl.kernel,
        out_type=[jax.ShapeDtypeStruct((CAP // 128, 128), jnp.int32),
                  jax.ShapeDtypeStruct((CAP // 128, 128), jnp.int32)],
        mesh=mesh,
        compiler_params=_SC_PARAMS,
        scratch_types=[pltpu.VMEM((SCH, 16), jnp.float32),
                       pltpu.VMEM((8, 128), jnp.int32),
                       pltpu.VMEM((8, 128), jnp.int32)],
        name="sc_extract",
    )
    def k(rc_hbm, src_hbm, dst_hbm, rc_v, sidx_v, didx_v):
        c = lax.axis_index("c")
        s = lax.axis_index("s")
        wid = s * 2 + c

        def body(i, carry):
            sc = wid * NSUP + i
            base = pl.multiple_of(sc * SCH, SCH)
            pltpu.sync_copy(rc_hbm.at[pl.ds(base, SCH)], rc_v)
            for j in range(8):
                for t in range(8):
                    rows = j * 128 + t * 16 + lax.iota(jnp.int32, 16)
                    sv = plsc.load_gather(
                        rc_v, [rows, jnp.zeros((16,), jnp.int32)])
                    dv = plsc.load_gather(
                        rc_v, [rows, jnp.ones((16,), jnp.int32)])
                    sidx_v[j, pl.ds(t * 16, 16)] = jnp.clip(
                        sv.astype(jnp.int32), 0, NPAD - 1)
                    didx_v[j, pl.ds(t * 16, 16)] = jnp.clip(
                        dv.astype(jnp.int32), 0, NPAD - 1)
            r8 = pl.multiple_of(sc * 8, 8)
            pltpu.sync_copy(sidx_v, src_hbm.at[pl.ds(r8, 8)])
            pltpu.sync_copy(didx_v, dst_hbm.at[pl.ds(r8, 8)])
            return carry

        lax.fori_loop(0, NSUP, body, 0)

    return k(rc)


def _sc_gather(table, idx2, D):
    mesh = plsc.VectorSubcoreMesh(**_MESH)

    @functools.partial(
        pl.kernel,
        out_type=jax.ShapeDtypeStruct((CAP, D), jnp.float32),
        mesh=mesh,
        compiler_params=_SC_PARAMS,
        scratch_types=[pltpu.VMEM((8, 128), jnp.int32),
                       pltpu.VMEM((SCH, D), jnp.float32),
                       pltpu.SemaphoreType.DMA],
        name=f"sc_gather_{D}",
    )
    def k(table_hbm, idx_hbm, out_hbm, idx_v, rows_v, sem):
        c = lax.axis_index("c")
        s = lax.axis_index("s")
        wid = s * 2 + c

        def body(ci, carry):
            base = pl.multiple_of((wid * NSUP + ci) * SCH, SCH)
            pltpu.sync_copy(idx_hbm.at[pl.ds(pl.multiple_of(base // 128, 8), 8)],
                            idx_v)
            cps = [pltpu.async_copy(table_hbm.at[idx_v.at[j]],
                                    rows_v.at[pl.ds(j * 128, 128)], sem)
                   for j in range(8)]
            for cp in cps:
                cp.wait()
            pltpu.sync_copy(rows_v, out_hbm.at[pl.ds(base, SCH)])
            return carry

        lax.fori_loop(0, NSUP, body, 0)

    return k(table, idx2)


def _sc_scatter(op, idx2, D, zeros_tbl):
    mesh = plsc.VectorSubcoreMesh(**_MESH)

    @functools.partial(
        pl.kernel,
        out_type=jax.ShapeDtypeStruct((2, NPAD, D), jnp.float32),
        mesh=mesh,
        compiler_params=_SC_PARAMS,
        scratch_types=[pltpu.VMEM((8, 128), jnp.int32),
                       pltpu.VMEM((SCH, D), jnp.float32),
                       pltpu.VMEM_SHARED((NPAD, D), jnp.float32)],
        name=f"sc_scatter_{D}",
    )
    def k(op_hbm, idx_hbm, zeros_hbm, out_hbm, idx_v, rows_v, acc_sh):
        c = lax.axis_index("c")
        s = lax.axis_index("s")
        row0 = s * ROWS_PER_TILE
        pltpu.sync_copy(zeros_hbm.at[pl.ds(row0, ROWS_PER_TILE)],
                        acc_sh.at[pl.ds(row0, ROWS_PER_TILE)])
        plsc.subcore_barrier()

        def body(ci, carry):
            base = pl.multiple_of(((c * 16 + s) * NSUP + ci) * SCH, SCH)
            pltpu.sync_copy(idx_hbm.at[pl.ds(pl.multiple_of(base // 128, 8), 8)],
                            idx_v)
            pltpu.sync_copy(op_hbm.at[pl.ds(base, SCH)], rows_v)
            for j in range(8):
                pltpu.sync_copy(rows_v.at[pl.ds(j * 128, 128)],
                                acc_sh.at[idx_v.at[j]], add=True)
            return carry

        lax.fori_loop(0, NSUP, body, 0)
        plsc.subcore_barrier()
        pltpu.sync_copy(acc_sh.at[pl.ds(row0, ROWS_PER_TILE)],
                        out_hbm.at[c, pl.ds(row0, ROWS_PER_TILE)])

    return k(op, idx2, zeros_tbl)


def _tc_bucket_matmul(gp, rc, W, blkk, blk_poff, blk_cnt, Din, Dout,
                      deg_col=False):
    Dtot = Dout + (16 if deg_col else 0)

    def body(bk_ref, po_ref, cn_ref, gp_ref, rc_ref, w_ref, op_ref):
        pid = pl.program_id(0)
        q = pid * BLK + lax.broadcasted_iota(jnp.int32, (BLK, 1), 0)
        valid = (q - po_ref[pid]) < cn_ref[pid]
        bbe = jnp.where(valid, rc_ref[:, 2:3], 0.0)
        x = gp_ref[...] * bbe
        y = jnp.dot(x, w_ref[0], preferred_element_type=jnp.float32)
        if deg_col:
            y = jnp.concatenate(
                [y, bbe, jnp.zeros((BLK, 15), jnp.float32)], axis=1)
        op_ref[...] = y

    grid_spec = pltpu.PrefetchScalarGridSpec(
        num_scalar_prefetch=3,
        grid=(NBLK,),
        in_specs=[pl.BlockSpec((BLK, Din), lambda j, bk, po, cn: (j, 0)),
                  pl.BlockSpec((BLK, 16), lambda j, bk, po, cn: (j, 0)),
                  pl.BlockSpec((1, Din, Dout),
                               lambda j, bk, po, cn: (bk[j], 0, 0))],
        out_specs=pl.BlockSpec((BLK, Dtot), lambda j, bk, po, cn: (j, 0)),
    )
    return pl.pallas_call(
        body,
        grid_spec=grid_spec,
        out_shape=jax.ShapeDtypeStruct((CAP, Dtot), jnp.float32),
    )(blkk, blk_poff, blk_cnt, gp, rc, W)


def _tc_post(part, degp, xt, root, bias, Din, Dout, Dpart):
    def body(p_ref, d_ref, x_ref, r_ref, b_ref, o_ref):
        spl = p_ref[0][:, :Dout] + p_ref[1][:, :Dout]
        deg = jnp.maximum(d_ref[0][:, 32:33] + d_ref[1][:, 32:33], 1.0)
        val = spl / deg + jnp.dot(x_ref[...], r_ref[...],
                                  preferred_element_type=jnp.float32) + b_ref[...]
        o_ref[...] = jnp.where(val > 0, val,
                               jnp.exp(jnp.minimum(val, 0.0)) - 1.0)

    return pl.pallas_call(
        body,
        grid=(NPAD // 256,),
        in_specs=[pl.BlockSpec((2, 256, Dpart), lambda j: (0, j, 0)),
                  pl.BlockSpec((2, 256, 48), lambda j: (0, j, 0)),
                  pl.BlockSpec((256, Din), lambda j: (j, 0)),
                  pl.BlockSpec((Din, Dout), lambda j: (0, 0)),
                  pl.BlockSpec((1, Dout), lambda j: (0, 0))],
        out_specs=pl.BlockSpec((256, Dout), lambda j: (j, 0)),
        out_shape=jax.ShapeDtypeStruct((NPAD, Dout), jnp.float32),
    )(part, degp, xt, root, bias)


def _tc_head(h, lw1, lb1, lw2p, lb2p):
    def body(h_ref, w1_ref, b1_ref, w2_ref, b2_ref, o_ref):
        t = jnp.dot(h_ref[...], w1_ref[...],
                    preferred_element_type=jnp.float32) + b1_ref[...]
        t = jnp.where(t > 0, t, jnp.exp(jnp.minimum(t, 0.0)) - 1.0)
        lg = jnp.dot(t, w2_ref[...],
                     preferred_element_type=jnp.float32) + b2_ref[...]
        col = lax.broadcasted_iota(jnp.int32, (256, NPAD), 1)
        lg = jnp.where(col < N, lg, -1e30)
        m = jnp.max(lg, axis=1, keepdims=True)
        z = lg - m
        lse = jnp.log(jnp.sum(jnp.exp(z), axis=1, keepdims=True))
        o_ref[...] = z - lse

    return pl.pallas_call(
        body,
        grid=(NPAD // 256,),
        in_specs=[pl.BlockSpec((256, 64), lambda j: (j, 0)),
                  pl.BlockSpec((64, 256), lambda j: (0, 0)),
                  pl.BlockSpec((1, 256), lambda j: (0, 0)),
                  pl.BlockSpec((256, NPAD), lambda j: (0, 0)),
                  pl.BlockSpec((1, NPAD), lambda j: (0, 0))],
        out_specs=pl.BlockSpec((256, NPAD), lambda j: (j, 0)),
        out_shape=jax.ShapeDtypeStruct((NPAD, NPAD), jnp.float32),
    )(h, lw1, lb1, lw2p, lb2p)


def kernel(x, edge_index, pseudo, W1, r1, b1, W2, r2, b2, W3, r3, b3,
           W4, r4, b4, W5, r5, b5, W6, r6, b6, lw1, lb1, lw2, lb2):
    src = edge_index[0].astype(jnp.int32)
    dst = edge_index[1].astype(jnp.int32)
    src_p = jnp.pad(src, (0, EPAD - E))
    dst_p = jnp.pad(dst, (0, EPAD - E))
    pseudo8 = jnp.pad(pseudo.astype(jnp.float32), ((0, EPAD - E), (0, 5)))

    b8, k8 = _basis(pseudo8)

    hist = _hist(k8)[:, 0, :]
    blockpre = (jnp.cumsum(hist, axis=0) - hist).reshape(NEB, 1, 128)
    counts = jnp.sum(hist, axis=0)
    pc = ((counts + BLK - 1) // BLK) * BLK
    pend = jnp.cumsum(pc)
    poff = (pend - pc).astype(jnp.int32)
    poff2 = poff.reshape(1, 128)

    jm = jnp.arange(NBLK, dtype=jnp.int32)[:, None] * BLK
    blkk = jnp.minimum(
        jnp.sum((jm >= pend[None, :K]).astype(jnp.int32), axis=1),
        K - 1).astype(jnp.int32)
    oh_b = (blkk[:, None] ==
            jnp.arange(K, dtype=jnp.int32)[None, :]).astype(jnp.int32)
    blk_poff = jnp.sum(oh_b * poff[None, :K], axis=1).astype(jnp.int32)
    blk_cnt = jnp.sum(oh_b * counts[None, :K], axis=1).astype(jnp.int32)

    srcf = src_p.astype(jnp.float32).reshape(EPAD, 1)
    dstf = dst_p.astype(jnp.float32).reshape(EPAD, 1)
    slots, recs = _slots_records(k8, b8, srcf, dstf, poff2, blockpre)
    slot2 = slots.reshape(P // 128, 128)
    rc = _rc_scatter(recs, slot2)
    src2, dst2 = _extract(rc)

    zeros48 = jnp.zeros((NPAD, 48), jnp.float32)
    zeros64 = jnp.zeros((NPAD, 64), jnp.float32)

    x1 = jnp.zeros((NPAD, 16), jnp.float32).at[:N, 0].set(x[:, 0])
    W1p = jnp.zeros((K, 16, 32), jnp.float32).at[:, :1, :].set(W1)
    r1p = jnp.zeros((16, 32), jnp.float32).at[0:1].set(r1)

    gp = _sc_gather(x1, src2, 16)
    op = _tc_bucket_matmul(gp, rc, W1p, blkk, blk_poff, blk_cnt, 16, 32,
                           deg_col=True)
    part1 = _sc_scatter(op, dst2, 48, zeros48)
    h = _tc_post(part1, part1, x1, r1p, b1.reshape(1, 32), 16, 32, 48)

    def layer(xt, W, root, bias, Din, Dout):
        gp = _sc_gather(xt, src2, Din)
        op = _tc_bucket_matmul(gp, rc, W, blkk, blk_poff, blk_cnt, Din, Dout)
        part = _sc_scatter(op, dst2, Dout, zeros64)
        return _tc_post(part, part1, xt, root, bias.reshape(1, Dout),
                        Din, Dout, Dout)

    h = jnp.pad(h, ((0, 0), (0, 32)))
    W2p = jnp.zeros((K, 64, 64), jnp.float32).at[:, :32, :].set(W2)
    r2p = jnp.zeros((64, 64), jnp.float32).at[:32].set(r2)
    h = layer(h, W2p, r2p, b2, 64, 64)
    h = layer(h, W3, r3, b3, 64, 64)
    h = layer(h, W4, r4, b4, 64, 64)
    h = layer(h, W5, r5, b5, 64, 64)
    h = layer(h, W6, r6, b6, 64, 64)

    lw2p = jnp.pad(lw2, ((0, 0), (0, NPAD - N)))
    lb2p = jnp.pad(lb2, (0, NPAD - N)).reshape(1, NPAD)
    out_full = _tc_head(h, lw1, lb1.reshape(1, 256), lw2p, lb2p)
    return out_full[:N, :N]
